# baseline (speedup 1.0000x reference)
import functools

import jax
import jax.numpy as jnp
from jax import lax
from jax.experimental import pallas as pl
from jax.experimental.pallas import tpu as pltpu


def _round_up(x, m):
    return (x + m - 1) // m * m


def _fused_kernel(idx_sm_ref, src_idx_ref, dst_idx_ref, src2_ref, ee_ref,
                  out_ref, deg_ref, gbuf_ref, *, n_dst, edge_tile, n_chunks):
    e_step = pl.program_id(0)

    @pl.when(e_step == 0)
    def _init():
        out_ref[...] = jnp.zeros_like(out_ref)
        deg_ref[...] = jnp.zeros_like(deg_ref)

    te = edge_tile
    stride = te + 1

    base = e_step * te
    for mi in range(te):
        i = pl.multiple_of(idx_sm_ref[base + mi], n_chunks)
        gbuf_ref[mi:mi + n_chunks * stride:stride, :] = (
            src2_ref[pl.ds(i, n_chunks), :])

    src_ids = src_idx_ref[...]
    valid = src_ids >= 0

    gathered = jnp.concatenate(
        [gbuf_ref[pl.ds(j * stride, te), :] for j in range(n_chunks)],
        axis=1)
    msg = gathered + ee_ref[...]
    msg = jnp.where(valid, msg, 0.0).astype(jnp.bfloat16)

    dst_ids = dst_idx_ref[...]
    dst_iota = lax.broadcasted_iota(jnp.int32, (n_dst, te), 0)
    m = dst_iota == dst_ids
    s = m.astype(jnp.bfloat16)

    out_ref[...] += jnp.dot(s, msg, preferred_element_type=jnp.float32)
    deg_ref[...] += jnp.sum(m, axis=1, keepdims=True).astype(jnp.float32)

    @pl.when(e_step == pl.num_programs(0) - 1)
    def _finalize():
        deg = deg_ref[...]
        inv = jnp.where(deg > 0.0, 1.0 / jnp.maximum(deg, 1.0), 0.0)
        out_ref[...] *= inv


def _agg_fused(src_idx, dst_idx, src_embedding, edge_embedding, num_dst_nodes,
               *, edge_tile=1024):
    E = edge_embedding.shape[0]
    n_src, D = src_embedding.shape

    D_pad = _round_up(D, 128)
    if D_pad != D:
        src_embedding = jnp.pad(src_embedding, ((0, 0), (0, D_pad - D)))
        edge_embedding = jnp.pad(edge_embedding, ((0, 0), (0, D_pad - D)))
    n_src_pad = _round_up(n_src, 8)
    if n_src_pad != n_src:
        src_embedding = jnp.pad(src_embedding, ((0, n_src_pad - n_src), (0, 0)))
    n_dst_pad = _round_up(max(num_dst_nodes, 1), 8)
    n_chunks = D_pad // 128

    E_pad = _round_up(max(E, 1), edge_tile)
    n_tiles = E_pad // edge_tile

    src_idx_p = jnp.full((E_pad, 1), -1, jnp.int32).at[:E, 0].set(
        src_idx.astype(jnp.int32))
    dst_idx_p = jnp.full((1, E_pad), -1, jnp.int32).at[0, :E].set(
        dst_idx.astype(jnp.int32))
    idx_sm = (jnp.maximum(src_idx_p[:, 0], 0) * n_chunks).astype(jnp.int32)
    src2 = src_embedding.reshape(n_src_pad * n_chunks, 128)

    stride = edge_tile + 1
    vmem_est = (
        n_src_pad * D_pad * 4
        + 2 * edge_tile * D_pad * 4
        + n_dst_pad * D_pad * 4
        + stride * n_chunks * 128 * 4
        + n_dst_pad * edge_tile * 2
        + edge_tile * D_pad * 8
    )
    cost = pl.CostEstimate(
        flops=2 * E_pad * D_pad * n_dst_pad,
        transcendentals=0,
        bytes_accessed=4 * (E * D_pad + n_dst_pad * D_pad + 2 * E_pad
                            + n_src_pad * D_pad),
    )
    grid_spec = pltpu.PrefetchScalarGridSpec(
        num_scalar_prefetch=1,
        grid=(n_tiles,),
        in_specs=[
            pl.BlockSpec((edge_tile, 1), lambda e, idx: (e, 0)),
            pl.BlockSpec((1, edge_tile), lambda e, idx: (0, e)),
            pl.BlockSpec((n_src_pad * n_chunks, 128), lambda e, idx: (0, 0),
                         pipeline_mode=pl.Buffered(1)),
            pl.BlockSpec((edge_tile, D_pad), lambda e, idx: (e, 0)),
        ],
        out_specs=pl.BlockSpec((n_dst_pad, D_pad), lambda e, idx: (0, 0)),
        scratch_shapes=[
            pltpu.VMEM((n_dst_pad, 1), jnp.float32),
            pltpu.VMEM((stride * n_chunks, 128), jnp.float32),
        ],
    )
    out = pl.pallas_call(
        functools.partial(_fused_kernel, n_dst=n_dst_pad, edge_tile=edge_tile,
                          n_chunks=n_chunks),
        out_shape=jax.ShapeDtypeStruct((n_dst_pad, D_pad), jnp.float32),
        grid_spec=grid_spec,
        compiler_params=pltpu.CompilerParams(
            dimension_semantics=("arbitrary",),
            vmem_limit_bytes=int(min(max(vmem_est + (16 << 20), 32 << 20),
                                     60 << 20)),
        ),
        cost_estimate=cost,
    )(idx_sm, src_idx_p, dst_idx_p, src2, edge_embedding)

    return out[:num_dst_nodes, :D]


def kernel(src_idx, dst_idx, src_embedding, edge_embedding):
    return _agg_fused(src_idx, dst_idx, src_embedding, edge_embedding, 4096,
                      edge_tile=3584)


# speedup vs baseline: 1.0169x; 1.0169x over previous
import functools

import jax
import jax.numpy as jnp
from jax import lax
from jax.experimental import pallas as pl
from jax.experimental.pallas import tpu as pltpu


def _round_up(x, m):
    return (x + m - 1) // m * m


def _fused_kernel(idx_sm_ref, src_idx_ref, dst_idx_ref, src2_ref, ee_ref,
                  out_ref, deg_ref, gbuf_ref, *, n_dst, edge_tile, n_chunks):
    e_step = pl.program_id(0)

    @pl.when(e_step == 0)
    def _init():
        out_ref[...] = jnp.zeros_like(out_ref)
        deg_ref[...] = jnp.zeros_like(deg_ref)

    te = edge_tile
    stride = te + 1

    base = e_step * te
    for mi in range(te):
        i = pl.multiple_of(idx_sm_ref[base + mi], n_chunks)
        gbuf_ref[mi:mi + n_chunks * stride:stride, :] = (
            src2_ref[pl.ds(i, n_chunks), :])

    src_ids = src_idx_ref[...]
    valid = src_ids >= 0

    gathered = jnp.concatenate(
        [gbuf_ref[pl.ds(j * stride, te), :] for j in range(n_chunks)],
        axis=1)
    msg = gathered + ee_ref[...]
    msg = jnp.where(valid, msg, 0.0).astype(jnp.bfloat16)

    dst_ids = dst_idx_ref[...]
    dst_iota = lax.broadcasted_iota(jnp.int32, (n_dst, te), 0)
    m = dst_iota == dst_ids
    s = m.astype(jnp.bfloat16)

    out_ref[...] += jnp.dot(s, msg, preferred_element_type=jnp.float32)
    deg_ref[...] += jnp.sum(m, axis=1, keepdims=True).astype(jnp.float32)

    @pl.when(e_step == pl.num_programs(0) - 1)
    def _finalize():
        deg = deg_ref[...]
        inv = jnp.where(deg > 0.0, 1.0 / jnp.maximum(deg, 1.0), 0.0)
        out_ref[...] *= inv


def _agg_fused(src_idx, dst_idx, src_embedding, edge_embedding, num_dst_nodes,
               *, edge_tile=1024):
    E = edge_embedding.shape[0]
    n_src, D = src_embedding.shape

    D_pad = _round_up(D, 128)
    if D_pad != D:
        src_embedding = jnp.pad(src_embedding, ((0, 0), (0, D_pad - D)))
        edge_embedding = jnp.pad(edge_embedding, ((0, 0), (0, D_pad - D)))
    n_src_pad = _round_up(n_src, 8)
    if n_src_pad != n_src:
        src_embedding = jnp.pad(src_embedding, ((0, n_src_pad - n_src), (0, 0)))
    n_dst_pad = _round_up(max(num_dst_nodes, 1), 8)
    n_chunks = D_pad // 128

    E_pad = _round_up(max(E, 1), edge_tile)
    n_tiles = E_pad // edge_tile

    src_idx_p = jnp.full((E_pad, 1), -1, jnp.int32).at[:E, 0].set(
        src_idx.astype(jnp.int32))
    dst_idx_p = jnp.full((1, E_pad), -1, jnp.int32).at[0, :E].set(
        dst_idx.astype(jnp.int32))
    idx_sm = (jnp.maximum(src_idx_p[:, 0], 0) * n_chunks).astype(jnp.int32)
    src2 = src_embedding.reshape(n_src_pad * n_chunks, 128)

    stride = edge_tile + 1
    vmem_est = (
        n_src_pad * D_pad * 4
        + 2 * edge_tile * D_pad * 4
        + n_dst_pad * D_pad * 4
        + stride * n_chunks * 128 * 4
        + n_dst_pad * edge_tile * 2
        + edge_tile * D_pad * 8
    )
    cost = pl.CostEstimate(
        flops=2 * E_pad * D_pad * n_dst_pad,
        transcendentals=0,
        bytes_accessed=4 * (E * D_pad + n_dst_pad * D_pad + 2 * E_pad
                            + n_src_pad * D_pad),
    )
    grid_spec = pltpu.PrefetchScalarGridSpec(
        num_scalar_prefetch=1,
        grid=(n_tiles,),
        in_specs=[
            pl.BlockSpec((edge_tile, 1), lambda e, idx: (e, 0)),
            pl.BlockSpec((1, edge_tile), lambda e, idx: (0, e)),
            pl.BlockSpec((n_src_pad * n_chunks, 128), lambda e, idx: (0, 0),
                         pipeline_mode=pl.Buffered(1)),
            pl.BlockSpec((edge_tile, D_pad), lambda e, idx: (e, 0)),
        ],
        out_specs=pl.BlockSpec((n_dst_pad, D_pad), lambda e, idx: (0, 0)),
        scratch_shapes=[
            pltpu.VMEM((n_dst_pad, 1), jnp.float32),
            pltpu.VMEM((stride * n_chunks, 128), jnp.float32),
        ],
    )
    out = pl.pallas_call(
        functools.partial(_fused_kernel, n_dst=n_dst_pad, edge_tile=edge_tile,
                          n_chunks=n_chunks),
        out_shape=jax.ShapeDtypeStruct((n_dst_pad, D_pad), jnp.float32),
        grid_spec=grid_spec,
        compiler_params=pltpu.CompilerParams(
            dimension_semantics=("arbitrary",),
            vmem_limit_bytes=int(min(max(vmem_est + (16 << 20), 32 << 20),
                                     60 << 20)),
        ),
        cost_estimate=cost,
    )(idx_sm, src_idx_p, dst_idx_p, src2, edge_embedding)

    return out[:num_dst_nodes, :D]


def kernel(src_idx, dst_idx, src_embedding, edge_embedding):
    return _agg_fused(src_idx, dst_idx, src_embedding, edge_embedding, 4096,
                      edge_tile=2816)


# speedup vs baseline: 1.0691x; 1.0513x over previous
import functools

import jax
import jax.numpy as jnp
from jax import lax
from jax.experimental import pallas as pl
from jax.experimental.pallas import tpu as pltpu


def _round_up(x, m):
    return (x + m - 1) // m * m


def _fused_kernel(idx_sm_ref, src_idx_ref, dst_idx_ref, src2_ref, ee_ref,
                  out_ref, deg_ref, gbuf_ref, *, n_dst, edge_tile, n_chunks):
    e_step = pl.program_id(0)

    @pl.when(e_step == 0)
    def _init():
        out_ref[...] = jnp.zeros_like(out_ref)
        deg_ref[...] = jnp.zeros_like(deg_ref)

    te = edge_tile
    stride = te + 1

    base = e_step * te
    for mi in range(te):
        i = pl.multiple_of(idx_sm_ref[base + mi], n_chunks)
        gbuf_ref[mi:mi + n_chunks * stride:stride, :] = (
            src2_ref[pl.ds(i, n_chunks), :])

    src_ids = src_idx_ref[...]
    valid = src_ids >= 0

    gathered = jnp.concatenate(
        [gbuf_ref[pl.ds(j * stride, te), :] for j in range(n_chunks)],
        axis=1)
    msg = gathered + ee_ref[...]
    msg = jnp.where(valid, msg, 0.0).astype(jnp.bfloat16)

    dst_ids = dst_idx_ref[...]
    nh = n_dst // 2
    for h in range(2):
        half_iota = lax.broadcasted_iota(jnp.int32, (nh, te), 0) + h * nh
        m = half_iota == dst_ids
        s = m.astype(jnp.bfloat16)
        out_ref[pl.ds(h * nh, nh), :] += jnp.dot(
            s, msg, preferred_element_type=jnp.float32)
        deg_ref[pl.ds(h * nh, nh), :] += jnp.sum(
            m, axis=1, keepdims=True).astype(jnp.float32)

    @pl.when(e_step == pl.num_programs(0) - 1)
    def _finalize():
        deg = deg_ref[...]
        inv = jnp.where(deg > 0.0, 1.0 / jnp.maximum(deg, 1.0), 0.0)
        out_ref[...] *= inv


def _agg_fused(src_idx, dst_idx, src_embedding, edge_embedding, num_dst_nodes,
               *, edge_tile=1024):
    E = edge_embedding.shape[0]
    n_src, D = src_embedding.shape

    D_pad = _round_up(D, 128)
    if D_pad != D:
        src_embedding = jnp.pad(src_embedding, ((0, 0), (0, D_pad - D)))
        edge_embedding = jnp.pad(edge_embedding, ((0, 0), (0, D_pad - D)))
    n_src_pad = _round_up(n_src, 8)
    if n_src_pad != n_src:
        src_embedding = jnp.pad(src_embedding, ((0, n_src_pad - n_src), (0, 0)))
    n_dst_pad = _round_up(max(num_dst_nodes, 1), 16)
    n_chunks = D_pad // 128

    E_pad = _round_up(max(E, 1), edge_tile)
    n_tiles = E_pad // edge_tile

    src_idx_p = jnp.full((E_pad, 1), -1, jnp.int32).at[:E, 0].set(
        src_idx.astype(jnp.int32))
    dst_idx_p = jnp.full((1, E_pad), -1, jnp.int32).at[0, :E].set(
        dst_idx.astype(jnp.int32))
    idx_sm = (jnp.maximum(src_idx_p[:, 0], 0) * n_chunks).astype(jnp.int32)
    src2 = src_embedding.reshape(n_src_pad * n_chunks, 128)

    stride = edge_tile + 1
    vmem_est = (
        n_src_pad * D_pad * 4
        + 2 * edge_tile * D_pad * 4
        + n_dst_pad * D_pad * 4
        + stride * n_chunks * 128 * 4
        + n_dst_pad * edge_tile * 2
        + edge_tile * D_pad * 8
    )
    cost = pl.CostEstimate(
        flops=2 * E_pad * D_pad * n_dst_pad,
        transcendentals=0,
        bytes_accessed=4 * (E * D_pad + n_dst_pad * D_pad + 2 * E_pad
                            + n_src_pad * D_pad),
    )
    grid_spec = pltpu.PrefetchScalarGridSpec(
        num_scalar_prefetch=1,
        grid=(n_tiles,),
        in_specs=[
            pl.BlockSpec((edge_tile, 1), lambda e, idx: (e, 0)),
            pl.BlockSpec((1, edge_tile), lambda e, idx: (0, e)),
            pl.BlockSpec((n_src_pad * n_chunks, 128), lambda e, idx: (0, 0),
                         pipeline_mode=pl.Buffered(1)),
            pl.BlockSpec((edge_tile, D_pad), lambda e, idx: (e, 0)),
        ],
        out_specs=pl.BlockSpec((n_dst_pad, D_pad), lambda e, idx: (0, 0)),
        scratch_shapes=[
            pltpu.VMEM((n_dst_pad, 1), jnp.float32),
            pltpu.VMEM((stride * n_chunks, 128), jnp.float32),
        ],
    )
    out = pl.pallas_call(
        functools.partial(_fused_kernel, n_dst=n_dst_pad, edge_tile=edge_tile,
                          n_chunks=n_chunks),
        out_shape=jax.ShapeDtypeStruct((n_dst_pad, D_pad), jnp.float32),
        grid_spec=grid_spec,
        compiler_params=pltpu.CompilerParams(
            dimension_semantics=("arbitrary",),
            vmem_limit_bytes=int(min(max(vmem_est + (16 << 20), 32 << 20),
                                     60 << 20)),
        ),
        cost_estimate=cost,
    )(idx_sm, src_idx_p, dst_idx_p, src2, edge_embedding)

    return out[:num_dst_nodes, :D]


def kernel(src_idx, dst_idx, src_embedding, edge_embedding):
    return _agg_fused(src_idx, dst_idx, src_embedding, edge_embedding, 4096,
                      edge_tile=2560)


# speedup vs baseline: 1.2062x; 1.1282x over previous
import functools

import jax
import jax.numpy as jnp
from jax import lax
from jax.experimental import pallas as pl
from jax.experimental.pallas import tpu as pltpu


def _round_up(x, m):
    return (x + m - 1) // m * m


def _fused_kernel(idx_sm_ref, src_idx_ref, dst_idx_ref, src2_ref, ee_ref,
                  out_ref, deg_ref, gbuf_ref, *, n_dst, edge_tile, n_chunks):
    e_step = pl.program_id(0)

    @pl.when(e_step == 0)
    def _init():
        out_ref[...] = jnp.zeros_like(out_ref)
        deg_ref[...] = jnp.zeros_like(deg_ref)

    te = edge_tile
    stride = te + 1

    base = e_step * te
    for mi in range(te):
        i = pl.multiple_of(idx_sm_ref[base + mi], n_chunks)
        gbuf_ref[mi:mi + n_chunks * stride:stride, :] = (
            src2_ref[pl.ds(i, n_chunks), :])

    src_ids = src_idx_ref[...]
    valid = src_ids >= 0

    gathered = jnp.concatenate(
        [gbuf_ref[pl.ds(j * stride, te), :] for j in range(n_chunks)],
        axis=1)
    msg = gathered + ee_ref[...]
    msg = jnp.where(valid, msg, 0.0).astype(jnp.bfloat16)

    dst_ids = dst_idx_ref[...]
    nh = n_dst // 4
    for h in range(4):
        half_iota = lax.broadcasted_iota(jnp.int32, (nh, te), 0) + h * nh
        m = half_iota == dst_ids
        s = m.astype(jnp.bfloat16)
        out_ref[pl.ds(h * nh, nh), :] += jnp.dot(
            s, msg, preferred_element_type=jnp.float32)
        deg_ref[pl.ds(h * nh, nh), :] += jnp.sum(
            m, axis=1, keepdims=True).astype(jnp.float32)

    @pl.when(e_step == pl.num_programs(0) - 1)
    def _finalize():
        deg = deg_ref[...]
        inv = jnp.where(deg > 0.0, 1.0 / jnp.maximum(deg, 1.0), 0.0)
        out_ref[...] *= inv


def _agg_fused(src_idx, dst_idx, src_embedding, edge_embedding, num_dst_nodes,
               *, edge_tile=1024):
    E = edge_embedding.shape[0]
    n_src, D = src_embedding.shape

    D_pad = _round_up(D, 128)
    if D_pad != D:
        src_embedding = jnp.pad(src_embedding, ((0, 0), (0, D_pad - D)))
        edge_embedding = jnp.pad(edge_embedding, ((0, 0), (0, D_pad - D)))
    n_src_pad = _round_up(n_src, 8)
    if n_src_pad != n_src:
        src_embedding = jnp.pad(src_embedding, ((0, n_src_pad - n_src), (0, 0)))
    n_dst_pad = _round_up(max(num_dst_nodes, 1), 32)
    n_chunks = D_pad // 128

    E_pad = _round_up(max(E, 1), edge_tile)
    n_tiles = E_pad // edge_tile

    src_idx_p = jnp.full((E_pad, 1), -1, jnp.int32).at[:E, 0].set(
        src_idx.astype(jnp.int32))
    dst_idx_p = jnp.full((1, E_pad), -1, jnp.int32).at[0, :E].set(
        dst_idx.astype(jnp.int32))
    idx_sm = (jnp.maximum(src_idx_p[:, 0], 0) * n_chunks).astype(jnp.int32)
    src2 = src_embedding.reshape(n_src_pad * n_chunks, 128)

    stride = edge_tile + 1
    vmem_est = (
        n_src_pad * D_pad * 4
        + 2 * edge_tile * D_pad * 4
        + n_dst_pad * D_pad * 4
        + stride * n_chunks * 128 * 4
        + n_dst_pad * edge_tile * 2
        + edge_tile * D_pad * 8
    )
    cost = pl.CostEstimate(
        flops=2 * E_pad * D_pad * n_dst_pad,
        transcendentals=0,
        bytes_accessed=4 * (E * D_pad + n_dst_pad * D_pad + 2 * E_pad
                            + n_src_pad * D_pad),
    )
    grid_spec = pltpu.PrefetchScalarGridSpec(
        num_scalar_prefetch=1,
        grid=(n_tiles,),
        in_specs=[
            pl.BlockSpec((edge_tile, 1), lambda e, idx: (e, 0)),
            pl.BlockSpec((1, edge_tile), lambda e, idx: (0, e)),
            pl.BlockSpec((n_src_pad * n_chunks, 128), lambda e, idx: (0, 0),
                         pipeline_mode=pl.Buffered(1)),
            pl.BlockSpec((edge_tile, D_pad), lambda e, idx: (e, 0)),
        ],
        out_specs=pl.BlockSpec((n_dst_pad, D_pad), lambda e, idx: (0, 0)),
        scratch_shapes=[
            pltpu.VMEM((n_dst_pad, 1), jnp.float32),
            pltpu.VMEM((stride * n_chunks, 128), jnp.float32),
        ],
    )
    out = pl.pallas_call(
        functools.partial(_fused_kernel, n_dst=n_dst_pad, edge_tile=edge_tile,
                          n_chunks=n_chunks),
        out_shape=jax.ShapeDtypeStruct((n_dst_pad, D_pad), jnp.float32),
        grid_spec=grid_spec,
        compiler_params=pltpu.CompilerParams(
            dimension_semantics=("arbitrary",),
            vmem_limit_bytes=int(min(max(vmem_est + (16 << 20), 32 << 20),
                                     60 << 20)),
        ),
        cost_estimate=cost,
    )(idx_sm, src_idx_p, dst_idx_p, src2, edge_embedding)

    return out[:num_dst_nodes, :D]


def kernel(src_idx, dst_idx, src_embedding, edge_embedding):
    return _agg_fused(src_idx, dst_idx, src_embedding, edge_embedding, 4096,
                      edge_tile=2560)
